# baseline (speedup 1.0000x reference)
import functools

import jax
import jax.numpy as jnp
from jax import lax
from jax.experimental import pallas as pl
from jax.experimental.pallas import tpu as pltpu
from jax.experimental.pallas import tpu_sc as plsc

_D = 128
_E = 2500
_NW = 32
_CHUNK = 125


def _sc_stage(table, gidx_t, sidx_t, n_seg, stage_table=False):
    n_chunks = gidx_t.shape[1]
    nt, w = table.shape
    rpt = n_seg // 16
    tpt = nt // 16
    mesh = plsc.VectorSubcoreMesh(core_axis_name="c", subcore_axis_name="s")

    @functools.partial(
        pl.kernel,
        out_type=jax.ShapeDtypeStruct((2, n_seg, w), jnp.float32),
        mesh=mesh,
        compiler_params=pltpu.CompilerParams(use_tc_tiling_on_sc=False),
        scratch_types=[
            pltpu.VMEM((n_chunks, _CHUNK), jnp.int32),
            pltpu.VMEM((n_chunks, _CHUNK), jnp.int32),
            [pltpu.VMEM((_CHUNK, w), jnp.float32) for _ in range(4)],
            pltpu.VMEM((16, w), jnp.float32),
            pltpu.VMEM_SHARED((n_seg, w), jnp.float32),
            (pltpu.VMEM_SHARED((nt, w), jnp.float32)
             if stage_table else pltpu.VMEM((8,), jnp.int32)),
            [pltpu.SemaphoreType.DMA for _ in range(4)],
            [pltpu.SemaphoreType.DMA for _ in range(4)],
            pltpu.SemaphoreType.DMA,
        ],
    )
    def stage(table_hbm, gidx_hbm, sidx_hbm, out_hbm,
              gidx_v, sidx_v, rows, zbuf_v, acc_sh, tab_sh, gsem, ssem, zsem):
        c = lax.axis_index("c")
        s = lax.axis_index("s")
        wid = s * 2 + c
        pltpu.sync_copy(gidx_hbm.at[wid], gidx_v)
        pltpu.sync_copy(sidx_hbm.at[wid], sidx_v)
        if stage_table:
            tb = s * tpt
            pltpu.sync_copy(table_hbm.at[pl.ds(tb, tpt)],
                            tab_sh.at[pl.ds(tb, tpt)])
            gather_src = tab_sh
        else:
            gather_src = table_hbm
        z = jnp.zeros((16,), jnp.float32)
        for i in range(16):
            for j in range(w // 16):
                zbuf_v[i, pl.ds(j * 16, 16)] = z
        base = s * rpt

        def zfire(r, carry):
            pltpu.async_copy(zbuf_v, acc_sh.at[pl.ds(base + r * 16, 16)], zsem)
            return carry

        def zdrain(r, carry):
            pltpu.make_async_copy(
                zbuf_v, acc_sh.at[pl.ds(base + r * 16, 16)], zsem).wait()
            return carry

        lax.fori_loop(0, rpt // 16, zfire, 0)
        lax.fori_loop(0, rpt // 16, zdrain, 0)
        plsc.subcore_barrier()

        def g_start(j, b):
            pltpu.async_copy(gather_src.at[gidx_v.at[j]], rows[b], gsem[b])

        def g_wait(j, b):
            pltpu.make_async_copy(
                gather_src.at[gidx_v.at[j]], rows[b], gsem[b]).wait()

        def s_start(j, b):
            pltpu.async_copy(rows[b], acc_sh.at[sidx_v.at[j]], ssem[b],
                             add=True)

        def s_wait(j, b):
            pltpu.make_async_copy(rows[b], acc_sh.at[sidx_v.at[j]],
                                  ssem[b]).wait()

        g_start(0, 0)
        g_start(1, 1)
        g_wait(0, 0)
        s_start(0, 0)
        g_start(2, 2)
        g_wait(1, 1)
        s_start(1, 1)
        g_start(3, 3)

        def body(q, carry):
            j0 = 2 + q * 4
            for o in range(4):
                j = j0 + o
                b = (2 + o) % 4
                g_wait(j, b)
                s_start(j, b)
                s_wait(j - 2, o)
                g_start(j + 2, o)
            return carry

        lax.fori_loop(0, (n_chunks - 4) // 4, body, 0)
        jl = n_chunks - 2
        g_wait(jl, jl % 4)
        s_start(jl, jl % 4)
        s_wait(jl - 2, (jl - 2) % 4)
        g_wait(jl + 1, (jl + 1) % 4)
        s_start(jl + 1, (jl + 1) % 4)
        s_wait(jl - 1, (jl - 1) % 4)
        s_wait(jl, jl % 4)
        s_wait(jl + 1, (jl + 1) % 4)
        plsc.subcore_barrier()
        pltpu.sync_copy(acc_sh.at[pl.ds(base, rpt)],
                        out_hbm.at[c, pl.ds(base, rpt)])

    return stage(table, gidx_t, sidx_t)


def _edge_tc(pe_lo, pe_hi, ce, W):
    ep = pe_lo.shape[1]
    hd = _D // 2

    def body(pl_ref, ph_ref, ce_ref, w_ref, lo_ref, hi_ref):
        cnt = jnp.maximum(ce_ref[0] + ce_ref[1], 1.0)
        es = jnp.concatenate(
            [pl_ref[0] + pl_ref[1], ph_ref[0] + ph_ref[1]], axis=1) / cnt
        ef = jnp.dot(es, w_ref[...], preferred_element_type=jnp.float32)
        lo_ref[...] = ef[:, :hd]
        hi_ref[...] = ef[:, hd:]

    return pl.pallas_call(
        body,
        out_shape=[jax.ShapeDtypeStruct((ep, hd), jnp.float32)] * 2,
    )(pe_lo, pe_hi, ce, W)


def _node_tc(pn, cn, b2, ybase, accin, s1, s2, w, s3):
    np_ = pn[0].shape[1]
    blk = 1024

    hd = _D // 2

    def body(pl_ref, ph_ref, cn_ref, b_ref, y_ref, a_ref, yo_ref, ao_ref):
        cnt = cn_ref[0] + cn_ref[1]
        inv = 1.0 / jnp.maximum(cnt, 1.0)
        has = jnp.where(cnt > 0.0, 1.0, 0.0)
        mean = jnp.concatenate(
            [pl_ref[0] + pl_ref[1], ph_ref[0] + ph_ref[1]], axis=1) * inv
        k = jnp.tanh(mean + has * b_ref[...])
        yo_ref[...] = y_ref[...] + s1 * k + s2 * a_ref[...]
        ao_ref[...] = s3 * a_ref[...] + w * k

    pn_lo, pn_hi = pn
    return pl.pallas_call(
        body,
        grid=(np_ // blk,),
        in_specs=[
            pl.BlockSpec((2, blk, hd), lambda i: (0, i, 0)),
            pl.BlockSpec((2, blk, hd), lambda i: (0, i, 0)),
            pl.BlockSpec((2, blk, 1), lambda i: (0, i, 0)),
            pl.BlockSpec((1, _D), lambda i: (0, 0)),
            pl.BlockSpec((blk, _D), lambda i: (i, 0)),
            pl.BlockSpec((blk, _D), lambda i: (i, 0)),
        ],
        out_specs=[
            pl.BlockSpec((blk, _D), lambda i: (i, 0)),
            pl.BlockSpec((blk, _D), lambda i: (i, 0)),
        ],
        out_shape=[jax.ShapeDtypeStruct((np_, _D), jnp.float32)] * 2,
    )(pn_lo, pn_hi, cn, b2, ybase, accin)


def kernel(node_features, node_idx, edge_idx, W, b):
    n, d = node_features.shape
    nnz = node_idx.shape[0]
    np_ = 10240
    ep = 2560
    n_chunks = nnz // (_NW * _CHUNK)
    dt = 0.25

    nidx_t = node_idx.astype(jnp.int32).reshape(_NW, n_chunks, _CHUNK)
    eidx_t = edge_idx.astype(jnp.int32).reshape(_NW, n_chunks, _CHUNK)
    ones_e = jnp.ones((ep, _D // 2), jnp.float32)

    ce = _sc_stage(ones_e, eidx_t, eidx_t, ep, True)[:, :, :1]
    cn = _sc_stage(ones_e, eidx_t, nidx_t, np_, True)[:, :, :1]

    y0 = jnp.pad(node_features, ((0, np_ - n), (0, 0)))
    b2 = b.reshape(1, _D)

    hd = _D // 2

    def drift(y_in):
        pe_lo = _sc_stage(y_in[:, :hd], nidx_t, eidx_t, ep, True)
        pe_hi = _sc_stage(y_in[:, hd:], nidx_t, eidx_t, ep, True)
        ef_lo, ef_hi = _edge_tc(pe_lo, pe_hi, ce, W)
        pn_lo = _sc_stage(ef_lo, eidx_t, nidx_t, np_, True)
        pn_hi = _sc_stage(ef_hi, eidx_t, nidx_t, np_, True)
        return pn_lo, pn_hi

    def step(y, _):
        pn1 = drift(y)
        y2, a1 = _node_tc(pn1, cn, b2, y, y, 0.5 * dt, 0.0, 1.0, 0.0)
        pn2 = drift(y2)
        y3, a2 = _node_tc(pn2, cn, b2, y, a1, 0.5 * dt, 0.0, 2.0, 1.0)
        pn3 = drift(y3)
        y4, a3 = _node_tc(pn3, cn, b2, y, a2, dt, 0.0, 2.0, 1.0)
        pn4 = drift(y4)
        y5, _ = _node_tc(pn4, cn, b2, y, a3, dt / 6.0, dt / 6.0, 0.0, 0.0)
        return y5, None

    yT, _ = lax.scan(step, y0, None, length=4)
    return yT[:n]


# speedup vs baseline: 1.2309x; 1.2309x over previous
import functools

import jax
import jax.numpy as jnp
from jax import lax
from jax.experimental import pallas as pl
from jax.experimental.pallas import tpu as pltpu
from jax.experimental.pallas import tpu_sc as plsc

_D = 128
_NW = 32
_CHUNK_S = 125
_CHUNK_B = 100
_MESH = dict(core_axis_name="c", subcore_axis_name="s")


def _zero_acc(zbuf_v, acc_sh, base, rows_per_tile, zsem, zrows):
    z = jnp.zeros((16,), jnp.float32)
    for i in range(zrows):
        for j in range(_D // 16):
            zbuf_v[i, pl.ds(j * 16, 16)] = z

    def zfire(r, carry):
        pltpu.async_copy(zbuf_v, acc_sh.at[pl.ds(base + r * zrows, zrows)],
                         zsem)
        return carry

    def zdrain(r, carry):
        pltpu.make_async_copy(
            zbuf_v, acc_sh.at[pl.ds(base + r * zrows, zrows)], zsem).wait()
        return carry

    lax.fori_loop(0, rows_per_tile // zrows, zfire, 0)
    lax.fori_loop(0, rows_per_tile // zrows, zdrain, 0)


def _sc_stage_small(table, gidx_t, sidx_t, n_seg):
    n_chunks = gidx_t.shape[1]
    rpt = n_seg // 16

    @functools.partial(
        pl.kernel,
        out_type=jax.ShapeDtypeStruct((2, n_seg, _D), jnp.float32),
        mesh=plsc.VectorSubcoreMesh(**_MESH),
        compiler_params=pltpu.CompilerParams(use_tc_tiling_on_sc=False),
        scratch_types=[
            pltpu.VMEM((n_chunks, _CHUNK_S), jnp.int32),
            pltpu.VMEM((n_chunks, _CHUNK_S), jnp.int32),
            [pltpu.VMEM((_CHUNK_S, _D), jnp.float32) for _ in range(4)],
            pltpu.VMEM((16, _D), jnp.float32),
            pltpu.VMEM_SHARED((n_seg, _D), jnp.float32),
            [pltpu.SemaphoreType.DMA for _ in range(4)],
            [pltpu.SemaphoreType.DMA for _ in range(4)],
            pltpu.SemaphoreType.DMA,
        ],
    )
    def stage(table_hbm, gidx_hbm, sidx_hbm, out_hbm,
              gidx_v, sidx_v, rows, zbuf_v, acc_sh, gsem, ssem, zsem):
        c = lax.axis_index("c")
        s = lax.axis_index("s")
        wid = s * 2 + c
        pltpu.sync_copy(gidx_hbm.at[wid], gidx_v)
        pltpu.sync_copy(sidx_hbm.at[wid], sidx_v)
        base = s * rpt
        _zero_acc(zbuf_v, acc_sh, base, rpt, zsem, 16)
        plsc.subcore_barrier()

        def g_start(j, b):
            pltpu.async_copy(table_hbm.at[gidx_v.at[j]], rows[b], gsem[b])

        def g_wait(j, b):
            pltpu.make_async_copy(
                table_hbm.at[gidx_v.at[j]], rows[b], gsem[b]).wait()

        def s_start(j, b):
            pltpu.async_copy(rows[b], acc_sh.at[sidx_v.at[j]], ssem[b],
                             add=True)

        def s_wait(j, b):
            pltpu.make_async_copy(rows[b], acc_sh.at[sidx_v.at[j]],
                                  ssem[b]).wait()

        g_start(0, 0)
        g_start(1, 1)
        g_wait(0, 0)
        s_start(0, 0)
        g_start(2, 2)
        g_wait(1, 1)
        s_start(1, 1)
        g_start(3, 3)

        def body(q, carry):
            j0 = 2 + q * 4
            for o in range(4):
                j = j0 + o
                b = (2 + o) % 4
                g_wait(j, b)
                s_start(j, b)
                s_wait(j - 2, o)
                g_start(j + 2, o)
            return carry

        lax.fori_loop(0, (n_chunks - 4) // 4, body, 0)
        jl = n_chunks - 2
        g_wait(jl, jl % 4)
        s_start(jl, jl % 4)
        s_wait(jl - 2, (jl - 2) % 4)
        g_wait(jl + 1, (jl + 1) % 4)
        s_start(jl + 1, (jl + 1) % 4)
        s_wait(jl - 1, (jl - 1) % 4)
        s_wait(jl, jl % 4)
        s_wait(jl + 1, (jl + 1) % 4)
        plsc.subcore_barrier()
        pltpu.sync_copy(acc_sh.at[pl.ds(base, rpt)],
                        out_hbm.at[c, pl.ds(base, rpt)])

    return stage(table, gidx_t, sidx_t)


def _sc_stage_big(table, gidx_t, sidx_t, n_seg):
    n_chunks = gidx_t.shape[1]
    half = n_chunks // 2
    rpt = n_seg // 16

    @functools.partial(
        pl.kernel,
        out_type=jax.ShapeDtypeStruct((2, n_seg, _D), jnp.float32),
        mesh=plsc.VectorSubcoreMesh(**_MESH),
        compiler_params=pltpu.CompilerParams(use_tc_tiling_on_sc=False),
        scratch_types=[
            pltpu.VMEM((half, _CHUNK_B), jnp.int32),
            pltpu.VMEM((half, _CHUNK_B), jnp.int32),
            [pltpu.VMEM((_CHUNK_B, _D), jnp.float32) for _ in range(2)],
            pltpu.VMEM((4, _D), jnp.float32),
            pltpu.VMEM_SHARED((n_seg, _D), jnp.float32),
            [pltpu.SemaphoreType.DMA for _ in range(2)],
            pltpu.SemaphoreType.DMA,
        ],
    )
    def stage(table_hbm, gidx_hbm, sidx_hbm, out_hbm,
              gidx_v, sidx_v, rows, zbuf_v, acc_sh, gsem, zsem):
        c = lax.axis_index("c")
        s = lax.axis_index("s")
        wid = s * 2 + c
        base = s * rpt
        _zero_acc(zbuf_v, acc_sh, base, rpt, zsem, 4)
        plsc.subcore_barrier()

        def g_start(j, b):
            pltpu.async_copy(table_hbm.at[gidx_v.at[j]], rows[b], gsem[b])

        def g_wait(j, b):
            pltpu.make_async_copy(
                table_hbm.at[gidx_v.at[j]], rows[b], gsem[b]).wait()

        def s_sync(j, b):
            pltpu.sync_copy(rows[b], acc_sh.at[sidx_v.at[j]], add=True)

        for h in range(2):
            pltpu.sync_copy(gidx_hbm.at[wid, pl.ds(h * half, half)], gidx_v)
            pltpu.sync_copy(sidx_hbm.at[wid, pl.ds(h * half, half)], sidx_v)
            g_start(0, 0)
            g_start(1, 1)

            def body(g, carry):
                j = g * 2
                g_wait(j, 0)
                s_sync(j, 0)
                g_start(j + 2, 0)
                g_wait(j + 1, 1)
                s_sync(j + 1, 1)
                g_start(j + 3, 1)
                return carry

            lax.fori_loop(0, (half - 2) // 2, body, 0)
            jl = half - 2
            g_wait(jl, 0)
            s_sync(jl, 0)
            g_wait(jl + 1, 1)
            s_sync(jl + 1, 1)
        plsc.subcore_barrier()
        pltpu.sync_copy(acc_sh.at[pl.ds(base, rpt)],
                        out_hbm.at[c, pl.ds(base, rpt)])

    return stage(table, gidx_t, sidx_t)


def _edge_tc(pe, ce, W):
    ep = pe.shape[1]

    def body(pe_ref, ce_ref, w_ref, out_ref):
        cnt = jnp.maximum(ce_ref[0] + ce_ref[1], 1.0)
        es = (pe_ref[0] + pe_ref[1]) / cnt
        out_ref[...] = jnp.dot(es, w_ref[...],
                               preferred_element_type=jnp.float32)

    return pl.pallas_call(
        body,
        out_shape=jax.ShapeDtypeStruct((ep, _D), jnp.float32),
    )(pe, ce, W)


def _node_tc(pn, cn, b2, ybase, accin, s1, s2, w, s3):
    np_ = pn.shape[1]
    blk = 1024

    def body(pn_ref, cn_ref, b_ref, y_ref, a_ref, yo_ref, ao_ref):
        cnt = cn_ref[0] + cn_ref[1]
        inv = 1.0 / jnp.maximum(cnt, 1.0)
        has = jnp.where(cnt > 0.0, 1.0, 0.0)
        k = jnp.tanh((pn_ref[0] + pn_ref[1]) * inv + has * b_ref[...])
        yo_ref[...] = y_ref[...] + s1 * k + s2 * a_ref[...]
        ao_ref[...] = s3 * a_ref[...] + w * k

    return pl.pallas_call(
        body,
        grid=(np_ // blk,),
        in_specs=[
            pl.BlockSpec((2, blk, _D), lambda i: (0, i, 0)),
            pl.BlockSpec((2, blk, 1), lambda i: (0, i, 0)),
            pl.BlockSpec((1, _D), lambda i: (0, 0)),
            pl.BlockSpec((blk, _D), lambda i: (i, 0)),
            pl.BlockSpec((blk, _D), lambda i: (i, 0)),
        ],
        out_specs=[
            pl.BlockSpec((blk, _D), lambda i: (i, 0)),
            pl.BlockSpec((blk, _D), lambda i: (i, 0)),
        ],
        out_shape=[jax.ShapeDtypeStruct((np_, _D), jnp.float32)] * 2,
    )(pn, cn, b2, ybase, accin)


def kernel(node_features, node_idx, edge_idx, W, b):
    n, d = node_features.shape
    nnz = node_idx.shape[0]
    np_ = 10240
    ep = 2560
    dt = 0.25

    ncs = nnz // (_NW * _CHUNK_S)
    ncb = nnz // (_NW * _CHUNK_B)
    nidx = node_idx.astype(jnp.int32)
    eidx = edge_idx.astype(jnp.int32)
    nidx_s = nidx.reshape(_NW, ncs, _CHUNK_S)
    eidx_s = eidx.reshape(_NW, ncs, _CHUNK_S)
    nidx_b = nidx.reshape(_NW, ncb, _CHUNK_B)
    eidx_b = eidx.reshape(_NW, ncb, _CHUNK_B)
    ones_n = jnp.ones((np_, _D), jnp.float32)

    ce = _sc_stage_small(ones_n, nidx_s, eidx_s, ep)[:, :, :1]
    cn = _sc_stage_big(ones_n, nidx_b, nidx_b, np_)[:, :, :1]

    y0 = jnp.pad(node_features, ((0, np_ - n), (0, 0)))
    b2 = b.reshape(1, _D)

    def drift(y_in):
        pe = _sc_stage_small(y_in, nidx_s, eidx_s, ep)
        ef = _edge_tc(pe, ce, W)
        return _sc_stage_big(ef, eidx_b, nidx_b, np_)

    def step(y, _):
        pn1 = drift(y)
        y2, a1 = _node_tc(pn1, cn, b2, y, y, 0.5 * dt, 0.0, 1.0, 0.0)
        pn2 = drift(y2)
        y3, a2 = _node_tc(pn2, cn, b2, y, a1, 0.5 * dt, 0.0, 2.0, 1.0)
        pn3 = drift(y3)
        y4, a3 = _node_tc(pn3, cn, b2, y, a2, dt, 0.0, 2.0, 1.0)
        pn4 = drift(y4)
        y5, _ = _node_tc(pn4, cn, b2, y, a3, dt / 6.0, dt / 6.0, 0.0, 0.0)
        return y5, None

    yT, _ = lax.scan(step, y0, None, length=4)
    return yT[:n]
